# baseline (speedup 1.0000x reference)
import functools

import jax
import jax.numpy as jnp
from jax import lax
from jax.experimental import pallas as pl
from jax.experimental.pallas import tpu as pltpu
from jax.experimental.pallas import tpu_sc as plsc

N = 10000
E = 320000
D = 128
L = 16
NC = 2
NS = 16
NW = NC * NS
C = 128
NCH = 80
E_PAD = NW * C * NCH
NP = 10112
NPA = 10016
ZROW = N
NRC = NP // C
RCT = -(-NRC // NS)
RC = 32
NRC2 = NPA // RC
RCT2 = -(-NRC2 // NS)

_mesh = plsc.VectorSubcoreMesh(core_axis_name="c", subcore_axis_name="s")


def _fill_f32(ref, rows, cols, val):
    v = jnp.full((L,), val, jnp.float32)

    def body(i, carry):
        for k in range(cols // L):
            ref[i, pl.ds(k * L, L)] = v
        return carry

    lax.fori_loop(0, rows, body, 0)


@functools.partial(
    pl.kernel,
    mesh=_mesh,
    out_type=[
        jax.ShapeDtypeStruct((NC, NP, D), jnp.float32),
    ],
    scratch_types=[
        pltpu.VMEM((2, C), jnp.int32),
        pltpu.VMEM((C, D), jnp.float32),
        pltpu.VMEM_SHARED((NP, D), jnp.float32),
        pltpu.SemaphoreType.DMA,
    ],
)
def _sc_sage(x_hbm, e_il, sum_out, idx_v, rows_v, accS, semG):
    cid = lax.axis_index("c")
    sid = lax.axis_index("s")
    wid = cid * NS + sid
    base = wid * NCH

    _fill_f32(rows_v, C, D, 0.0)
    for t in range(RCT):
        ch = sid + t * NS
        @pl.when(ch < NRC)
        def _():
            r = pl.multiple_of(ch * C, C)
            pltpu.sync_copy(rows_v, accS.at[pl.ds(r, C), :])
    plsc.subcore_barrier()

    def chunk(j, carry):
        pltpu.sync_copy(e_il.at[base + j], idx_v)
        pltpu.async_copy(x_hbm.at[idx_v.at[0]], rows_v, semG).wait()
        pltpu.sync_copy(rows_v, accS.at[idx_v.at[1]], add=True)
        return carry

    lax.fori_loop(0, NCH, chunk, 0)
    plsc.subcore_barrier()

    for t in range(RCT):
        ch = sid + t * NS
        @pl.when(ch < NRC)
        def _():
            r = pl.multiple_of(ch * C, C)
            pltpu.sync_copy(accS.at[pl.ds(r, C), :], sum_out.at[cid, pl.ds(r, C), :])


@functools.partial(
    pl.kernel,
    mesh=_mesh,
    out_type=[
        jax.ShapeDtypeStruct((NC, NP, D), jnp.float32),
    ],
    scratch_types=[
        pltpu.VMEM((2, 2, C), jnp.int32),
        pltpu.VMEM((C,), jnp.int32),
        pltpu.VMEM((C, D), jnp.float32),
        pltpu.VMEM((C, D), jnp.float32),
        pltpu.VMEM_SHARED((NP, D), jnp.float32),
        pltpu.SemaphoreType.DMA((2,)),
    ],
)
def _sc_hist(e_il, hist_out, idx2, srcp_v, onesA_v, onesB_v, histS, semI):
    cid = lax.axis_index("c")
    sid = lax.axis_index("s")
    wid = cid * NS + sid
    base = wid * NCH

    _fill_f32(onesA_v, C, D, 0.0)
    _fill_f32(onesB_v, C, D, 0.0)
    for t in range(RCT):
        ch = sid + t * NS
        @pl.when(ch < NRC)
        def _():
            r = pl.multiple_of(ch * C, C)
            pltpu.sync_copy(onesA_v, histS.at[pl.ds(r, C), :])
    one = jnp.full((L,), 1.0, jnp.float32)

    def setrow(i, carry):
        onesA_v[i, pl.ds(0, L)] = one
        onesB_v[i, pl.ds(L, L)] = one
        return carry

    lax.fori_loop(0, C, setrow, 0)
    plsc.subcore_barrier()

    def chunk(j, carry):
        @pl.when(j < NCH)
        def _():
            p = jnp.bitwise_and(j, 1)
            pltpu.async_copy(e_il.at[base + j], idx2.at[p], semI.at[p])
        @pl.when(j > 0)
        def _():
            q = jnp.bitwise_and(j - 1, 1)
            pltpu.make_async_copy(e_il.at[base], idx2.at[q], semI.at[q]).wait()
            for k in range(C // L):
                s = idx2[q, 0, pl.ds(k * L, L)]
                d = idx2[q, 1, pl.ds(k * L, L)]
                srcp_v[pl.ds(k * L, L)] = jnp.where(s == d, ZROW, s)
            pltpu.sync_copy(onesA_v, histS.at[idx2.at[q, 1]], add=True)
            pltpu.sync_copy(onesB_v, histS.at[srcp_v], add=True)
        return carry

    lax.fori_loop(0, NCH + 1, chunk, 0)
    plsc.subcore_barrier()

    for t in range(RCT):
        ch = sid + t * NS
        @pl.when(ch < NRC)
        def _():
            r = pl.multiple_of(ch * C, C)
            pltpu.sync_copy(histS.at[pl.ds(r, C), :], hist_out.at[cid, pl.ds(r, C), :])


@functools.partial(
    pl.kernel,
    mesh=_mesh,
    out_type=[jax.ShapeDtypeStruct((NC, NP, D), jnp.float32)],
    scratch_types=[
        pltpu.VMEM((2, C), jnp.int32),
        pltpu.VMEM((C, D), jnp.float32),
        pltpu.VMEM_SHARED((NP, D), jnp.float32),
        pltpu.SemaphoreType.DMA,
    ],
)
def _sc_cheb(h2_hbm, e_il, s2_out, idx_v, rows_v, accS, semG):
    cid = lax.axis_index("c")
    sid = lax.axis_index("s")
    wid = cid * NS + sid
    base = wid * NCH

    _fill_f32(rows_v, C, D, 0.0)
    for t in range(RCT):
        ch = sid + t * NS
        @pl.when(ch < NRC)
        def _():
            r = pl.multiple_of(ch * C, C)
            pltpu.sync_copy(rows_v, accS.at[pl.ds(r, C), :])
    plsc.subcore_barrier()

    def chunk(j, carry):
        pltpu.sync_copy(e_il.at[base + j], idx_v)
        for k in range(C // L):
            s = idx_v[0, pl.ds(k * L, L)]
            d = idx_v[1, pl.ds(k * L, L)]
            idx_v[0, pl.ds(k * L, L)] = jnp.where(s == d, ZROW, s)
        pltpu.async_copy(h2_hbm.at[idx_v.at[0]], rows_v, semG).wait()
        pltpu.sync_copy(rows_v, accS.at[idx_v.at[1]], add=True)
        return carry

    lax.fori_loop(0, NCH, chunk, 0)
    plsc.subcore_barrier()

    for t in range(RCT):
        ch = sid + t * NS
        @pl.when(ch < NRC)
        def _():
            r = pl.multiple_of(ch * C, C)
            pltpu.sync_copy(accS.at[pl.ds(r, C), :], s2_out.at[cid, pl.ds(r, C), :])


def _tc1_body(sum_ref, hist_ref, x_ref, wl_ref, wr_ref, b_ref, a_ref,
              h_ref, h2_ref, nd_ref):
    s = sum_ref[0] + sum_ref[1]
    hist = hist_ref[0] + hist_ref[1]
    cnt = hist[:, 0:1]
    mean = s / jnp.maximum(cnt, 1.0)
    out = mean @ wl_ref[...] + x_ref[...] @ wr_ref[...] + b_ref[...]
    nrm = jnp.sqrt(jnp.sum(out * out, axis=1, keepdims=True))
    out = out / jnp.maximum(nrm, 1e-12)
    a = a_ref[...]
    h = jnp.where(out > 0, out, a * out)
    deg = hist[:, L:L + 1]
    dinv = jnp.where(deg > 0, lax.rsqrt(deg), 0.0)
    h_ref[...] = h
    h2_ref[...] = dinv * h
    nd_ref[...] = -dinv


def _tc2_body(s2_ref, h_ref, nd_ref, w0_ref, w1_ref, b_ref, a_ref, o_ref):
    tx1 = (s2_ref[0] + s2_ref[1]) * nd_ref[...]
    out2 = h_ref[...] @ w0_ref[...] + tx1 @ w1_ref[...] + b_ref[...]
    a = a_ref[...]
    o_ref[...] = jnp.where(out2 > 0, out2, a * out2)


_BR = 632


def _tc1(sums, hists, x_pad, W_l, W_r, b, a):
    grid = (NP // _BR,)
    full = lambda i: (0, 0)
    return pl.pallas_call(
        _tc1_body,
        grid=grid,
        in_specs=[
            pl.BlockSpec((NC, _BR, D), lambda i: (0, i, 0)),
            pl.BlockSpec((NC, _BR, D), lambda i: (0, i, 0)),
            pl.BlockSpec((_BR, D), lambda i: (i, 0)),
            pl.BlockSpec((D, D), full),
            pl.BlockSpec((D, D), full),
            pl.BlockSpec((1, D), full),
            pl.BlockSpec((1, D), full),
        ],
        out_specs=[
            pl.BlockSpec((_BR, D), lambda i: (i, 0)),
            pl.BlockSpec((_BR, D), lambda i: (i, 0)),
            pl.BlockSpec((_BR, 1), lambda i: (i, 0)),
        ],
        out_shape=[
            jax.ShapeDtypeStruct((NP, D), jnp.float32),
            jax.ShapeDtypeStruct((NP, D), jnp.float32),
            jax.ShapeDtypeStruct((NP, 1), jnp.float32),
        ],
    )(sums, hists, x_pad, W_l, W_r, b, a)


def _tc2(s2, h, nd, W0, W1, b, a):
    grid = (NP // _BR,)
    full = lambda i: (0, 0)
    return pl.pallas_call(
        _tc2_body,
        grid=grid,
        in_specs=[
            pl.BlockSpec((NC, _BR, D), lambda i: (0, i, 0)),
            pl.BlockSpec((_BR, D), lambda i: (i, 0)),
            pl.BlockSpec((_BR, 1), lambda i: (i, 0)),
            pl.BlockSpec((D, D), full),
            pl.BlockSpec((D, D), full),
            pl.BlockSpec((1, D), full),
            pl.BlockSpec((1, D), full),
        ],
        out_specs=pl.BlockSpec((_BR, D), lambda i: (i, 0)),
        out_shape=jax.ShapeDtypeStruct((NP, D), jnp.float32),
    )(s2, h, nd, W0, W1, b, a)


def kernel(x, edge_index, W_sage_l, b_sage_l, W_sage_r, W_cheb0, W_cheb1,
           b_cheb, prelu_a):
    src = edge_index[0]
    dst = edge_index[1]
    pad = E_PAD - E
    fill = jnp.full((pad,), ZROW, jnp.int32)
    src2d = jnp.concatenate([src, fill]).reshape(NW * NCH, C)
    dst2d = jnp.concatenate([dst, fill]).reshape(NW * NCH, C)
    e_il = jnp.stack([src2d, dst2d], axis=1)
    x_pad = jnp.pad(x, ((0, NP - N), (0, 0)))

    (sums,) = _sc_sage(x_pad, e_il)
    (hists,) = _sc_hist(e_il)
    h, h2, nd = _tc1(sums, hists, x_pad,
                     W_sage_l, W_sage_r, b_sage_l.reshape(1, D),
                     prelu_a.reshape(1, D))
    (s2,) = _sc_cheb(h2, e_il)
    out2 = _tc2(s2, h, nd, W_cheb0, W_cheb1, b_cheb.reshape(1, D),
                prelu_a.reshape(1, D))
    return out2[:N]


# speedup vs baseline: 1.2103x; 1.2103x over previous
import functools

import jax
import jax.numpy as jnp
from jax import lax
from jax.experimental import pallas as pl
from jax.experimental.pallas import tpu as pltpu
from jax.experimental.pallas import tpu_sc as plsc

N = 10000
E = 320000
D = 128
L = 16
NC = 2
NS = 16
NW = NC * NS
C = 128
NCH = 80
E_PAD = NW * C * NCH
NP = 10112
NPA = 10016
ZROW = N
NRC = NP // C
RCT = -(-NRC // NS)
RC = 32
NRC2 = NPA // RC
RCT2 = -(-NRC2 // NS)

_mesh = plsc.VectorSubcoreMesh(core_axis_name="c", subcore_axis_name="s")


def _fill_f32(ref, rows, cols, val):
    v = jnp.full((L,), val, jnp.float32)

    def body(i, carry):
        for k in range(cols // L):
            ref[i, pl.ds(k * L, L)] = v
        return carry

    lax.fori_loop(0, rows, body, 0)


@functools.partial(
    pl.kernel,
    mesh=_mesh,
    out_type=[
        jax.ShapeDtypeStruct((NC, NP, D), jnp.float32),
    ],
    scratch_types=[
        pltpu.VMEM((C,), jnp.int32),
        pltpu.VMEM((C,), jnp.int32),
        pltpu.VMEM((C, D), jnp.float32),
        pltpu.VMEM_SHARED((NP, D), jnp.float32),
        pltpu.SemaphoreType.DMA,
    ],
)
def _sc_sage(x_hbm, src_hbm, dst_hbm, sum_out, src_v, dst_v, rows_v, accS, semG):
    cid = lax.axis_index("c")
    sid = lax.axis_index("s")
    wid = cid * NS + sid
    base = wid * (NCH * C)

    _fill_f32(rows_v, C, D, 0.0)
    for t in range(RCT):
        ch = sid + t * NS
        @pl.when(ch < NRC)
        def _():
            r = pl.multiple_of(ch * C, C)
            pltpu.sync_copy(rows_v, accS.at[pl.ds(r, C), :])
    plsc.subcore_barrier()

    def chunk(j, carry):
        ebase = pl.multiple_of(base + j * C, C)
        pltpu.sync_copy(src_hbm.at[pl.ds(ebase, C)], src_v)
        pltpu.sync_copy(dst_hbm.at[pl.ds(ebase, C)], dst_v)
        pltpu.async_copy(x_hbm.at[src_v], rows_v, semG).wait()
        pltpu.sync_copy(rows_v, accS.at[dst_v], add=True)
        return carry

    lax.fori_loop(0, NCH, chunk, 0)
    plsc.subcore_barrier()

    for t in range(RCT):
        ch = sid + t * NS
        @pl.when(ch < NRC)
        def _():
            r = pl.multiple_of(ch * C, C)
            pltpu.sync_copy(accS.at[pl.ds(r, C), :], sum_out.at[cid, pl.ds(r, C), :])


@functools.partial(
    pl.kernel,
    mesh=_mesh,
    out_type=[
        jax.ShapeDtypeStruct((NC, NP, D), jnp.float32),
    ],
    scratch_types=[
        pltpu.VMEM((2, 2, C), jnp.int32),
        pltpu.VMEM((C,), jnp.int32),
        pltpu.VMEM((C, D), jnp.float32),
        pltpu.VMEM((C, D), jnp.float32),
        pltpu.VMEM_SHARED((NP, D), jnp.float32),
        pltpu.SemaphoreType.DMA((2,)),
    ],
)
def _sc_hist(e_il, hist_out, idx2, srcp_v, onesA_v, onesB_v, histS, semI):
    cid = lax.axis_index("c")
    sid = lax.axis_index("s")
    wid = cid * NS + sid
    base = wid * NCH

    _fill_f32(onesA_v, C, D, 0.0)
    _fill_f32(onesB_v, C, D, 0.0)
    for t in range(RCT):
        ch = sid + t * NS
        @pl.when(ch < NRC)
        def _():
            r = pl.multiple_of(ch * C, C)
            pltpu.sync_copy(onesA_v, histS.at[pl.ds(r, C), :])
    one = jnp.full((L,), 1.0, jnp.float32)

    def setrow(i, carry):
        onesA_v[i, pl.ds(0, L)] = one
        onesB_v[i, pl.ds(L, L)] = one
        return carry

    lax.fori_loop(0, C, setrow, 0)
    plsc.subcore_barrier()

    def chunk(j, carry):
        @pl.when(j < NCH)
        def _():
            p = jnp.bitwise_and(j, 1)
            pltpu.async_copy(e_il.at[base + j], idx2.at[p], semI.at[p])
        @pl.when(j > 0)
        def _():
            q = jnp.bitwise_and(j - 1, 1)
            pltpu.make_async_copy(e_il.at[base], idx2.at[q], semI.at[q]).wait()
            for k in range(C // L):
                s = idx2[q, 0, pl.ds(k * L, L)]
                d = idx2[q, 1, pl.ds(k * L, L)]
                srcp_v[pl.ds(k * L, L)] = jnp.where(s == d, ZROW, s)
            pltpu.sync_copy(onesA_v, histS.at[idx2.at[q, 1]], add=True)
            pltpu.sync_copy(onesB_v, histS.at[srcp_v], add=True)
        return carry

    lax.fori_loop(0, NCH + 1, chunk, 0)
    plsc.subcore_barrier()

    for t in range(RCT):
        ch = sid + t * NS
        @pl.when(ch < NRC)
        def _():
            r = pl.multiple_of(ch * C, C)
            pltpu.sync_copy(histS.at[pl.ds(r, C), :], hist_out.at[cid, pl.ds(r, C), :])


@functools.partial(
    pl.kernel,
    mesh=_mesh,
    out_type=[jax.ShapeDtypeStruct((NC, NP, D), jnp.float32)],
    scratch_types=[
        pltpu.VMEM((C,), jnp.int32),
        pltpu.VMEM((C,), jnp.int32),
        pltpu.VMEM((C, D), jnp.float32),
        pltpu.VMEM_SHARED((NP, D), jnp.float32),
        pltpu.SemaphoreType.DMA,
    ],
)
def _sc_cheb(h2_hbm, src_hbm, dst_hbm, s2_out, src_v, dst_v, rows_v, accS, semG):
    cid = lax.axis_index("c")
    sid = lax.axis_index("s")
    wid = cid * NS + sid
    base = wid * NCH

    _fill_f32(rows_v, C, D, 0.0)
    for t in range(RCT):
        ch = sid + t * NS
        @pl.when(ch < NRC)
        def _():
            r = pl.multiple_of(ch * C, C)
            pltpu.sync_copy(rows_v, accS.at[pl.ds(r, C), :])
    plsc.subcore_barrier()

    def chunk(j, carry):
        ebase = pl.multiple_of(base + j * C, C)
        pltpu.sync_copy(src_hbm.at[pl.ds(ebase, C)], src_v)
        pltpu.sync_copy(dst_hbm.at[pl.ds(ebase, C)], dst_v)
        for k in range(C // L):
            s = src_v[pl.ds(k * L, L)]
            d = dst_v[pl.ds(k * L, L)]
            src_v[pl.ds(k * L, L)] = jnp.where(s == d, ZROW, s)
        pltpu.async_copy(h2_hbm.at[src_v], rows_v, semG).wait()
        pltpu.sync_copy(rows_v, accS.at[dst_v], add=True)
        return carry

    lax.fori_loop(0, NCH, chunk, 0)
    plsc.subcore_barrier()

    for t in range(RCT):
        ch = sid + t * NS
        @pl.when(ch < NRC)
        def _():
            r = pl.multiple_of(ch * C, C)
            pltpu.sync_copy(accS.at[pl.ds(r, C), :], s2_out.at[cid, pl.ds(r, C), :])


def _tc1_body(sum_ref, hist_ref, x_ref, wl_ref, wr_ref, b_ref, a_ref,
              h_ref, h2_ref, nd_ref):
    s = sum_ref[0] + sum_ref[1]
    hist = hist_ref[0] + hist_ref[1]
    cnt = hist[:, 0:1]
    mean = s / jnp.maximum(cnt, 1.0)
    out = mean @ wl_ref[...] + x_ref[...] @ wr_ref[...] + b_ref[...]
    nrm = jnp.sqrt(jnp.sum(out * out, axis=1, keepdims=True))
    out = out / jnp.maximum(nrm, 1e-12)
    a = a_ref[...]
    h = jnp.where(out > 0, out, a * out)
    deg = hist[:, L:L + 1]
    dinv = jnp.where(deg > 0, lax.rsqrt(deg), 0.0)
    h_ref[...] = h
    h2_ref[...] = dinv * h
    nd_ref[...] = -dinv


def _tc2_body(s2_ref, h_ref, nd_ref, w0_ref, w1_ref, b_ref, a_ref, o_ref):
    tx1 = (s2_ref[0] + s2_ref[1]) * nd_ref[...]
    out2 = h_ref[...] @ w0_ref[...] + tx1 @ w1_ref[...] + b_ref[...]
    a = a_ref[...]
    o_ref[...] = jnp.where(out2 > 0, out2, a * out2)


_BR = 632


def _tc1(sums, hists, x_pad, W_l, W_r, b, a):
    grid = (NP // _BR,)
    full = lambda i: (0, 0)
    return pl.pallas_call(
        _tc1_body,
        grid=grid,
        in_specs=[
            pl.BlockSpec((NC, _BR, D), lambda i: (0, i, 0)),
            pl.BlockSpec((NC, _BR, D), lambda i: (0, i, 0)),
            pl.BlockSpec((_BR, D), lambda i: (i, 0)),
            pl.BlockSpec((D, D), full),
            pl.BlockSpec((D, D), full),
            pl.BlockSpec((1, D), full),
            pl.BlockSpec((1, D), full),
        ],
        out_specs=[
            pl.BlockSpec((_BR, D), lambda i: (i, 0)),
            pl.BlockSpec((_BR, D), lambda i: (i, 0)),
            pl.BlockSpec((_BR, 1), lambda i: (i, 0)),
        ],
        out_shape=[
            jax.ShapeDtypeStruct((NP, D), jnp.float32),
            jax.ShapeDtypeStruct((NP, D), jnp.float32),
            jax.ShapeDtypeStruct((NP, 1), jnp.float32),
        ],
    )(sums, hists, x_pad, W_l, W_r, b, a)


def _tc2(s2, h, nd, W0, W1, b, a):
    grid = (NP // _BR,)
    full = lambda i: (0, 0)
    return pl.pallas_call(
        _tc2_body,
        grid=grid,
        in_specs=[
            pl.BlockSpec((NC, _BR, D), lambda i: (0, i, 0)),
            pl.BlockSpec((_BR, D), lambda i: (i, 0)),
            pl.BlockSpec((_BR, 1), lambda i: (i, 0)),
            pl.BlockSpec((D, D), full),
            pl.BlockSpec((D, D), full),
            pl.BlockSpec((1, D), full),
            pl.BlockSpec((1, D), full),
        ],
        out_specs=pl.BlockSpec((_BR, D), lambda i: (i, 0)),
        out_shape=jax.ShapeDtypeStruct((NP, D), jnp.float32),
    )(s2, h, nd, W0, W1, b, a)


def kernel(x, edge_index, W_sage_l, b_sage_l, W_sage_r, W_cheb0, W_cheb1,
           b_cheb, prelu_a):
    src = edge_index[0]
    dst = edge_index[1]
    pad = E_PAD - E
    fill = jnp.full((pad,), ZROW, jnp.int32)
    src2d = jnp.concatenate([src, fill]).reshape(NW * NCH, C)
    dst2d = jnp.concatenate([dst, fill]).reshape(NW * NCH, C)
    e_il = jnp.stack([src2d, dst2d], axis=1)
    x_pad = jnp.pad(x, ((0, NP - N), (0, 0)))

    src_p = jnp.concatenate([src, fill])
    dst_p = jnp.concatenate([dst, fill])
    (sums,) = _sc_sage(x_pad, src_p, dst_p)
    (hists,) = _sc_hist(e_il)
    h, h2, nd = _tc1(sums, hists, x_pad,
                     W_sage_l, W_sage_r, b_sage_l.reshape(1, D),
                     prelu_a.reshape(1, D))
    (s2,) = _sc_cheb(h2, src_p, dst_p)
    out2 = _tc2(s2, h, nd, W_cheb0, W_cheb1, b_cheb.reshape(1, D),
                prelu_a.reshape(1, D))
    return out2[:N]
